# baseline (speedup 1.0000x reference)
import functools

import jax
import jax.numpy as jnp
from jax import lax
from jax.experimental import pallas as pl
from jax.experimental.pallas import tpu as pltpu
from jax.experimental.pallas import tpu_sc as plsc

N_NODES = 10000
D = 128
NPAD = 10240
NC = 2
NS = 16
NW = NC * NS
CH = 64
TPW = NPAD
EPAD = NW * TPW
NCH0 = 320
NCH1 = 0
CHOFF1 = NS * NCH0
NB = 4
SEG = 16
XB = SEG - NB
ROWS_PER_TILE = NPAD // NS

_mesh = plsc.VectorSubcoreMesh(core_axis_name="c", subcore_axis_name="s")
_sc_params = pltpu.CompilerParams(needs_layout_passes=False)


@functools.partial(
    pl.kernel,
    out_type=jax.ShapeDtypeStruct((NC, NS, NPAD), jnp.float32),
    mesh=_mesh,
    compiler_params=_sc_params,
    scratch_types=[
        pltpu.VMEM((TPW,), jnp.int32),
        pltpu.VMEM((NPAD,), jnp.float32),
    ],
)
def _deg_kernel(dst_hbm, out_hbm, idx_v, hist_v):
  c = lax.axis_index("c")
  s = lax.axis_index("s")
  w = s * NC + c
  pltpu.sync_copy(dst_hbm.at[pl.ds(w * TPW, TPW)], idx_v)
  zeros = jnp.zeros((16,), jnp.float32)

  def zbody(i, carry):
    hist_v[pl.ds(i * 16, 16)] = zeros
    return carry

  lax.fori_loop(0, NPAD // 16, zbody, 0)
  ones = jnp.ones((16,), jnp.float32)

  def body(i, carry):
    idx = idx_v[pl.ds(i * 16, 16)]
    plsc.addupdate_scatter(hist_v, [idx], ones)
    return carry

  lax.fori_loop(0, TPW // 16, body, 0)
  pltpu.sync_copy(hist_v, out_hbm.at[c, s])


@functools.partial(
    pl.kernel,
    out_type=jax.ShapeDtypeStruct((NC, NPAD, D), jnp.float32),
    mesh=_mesh,
    compiler_params=_sc_params,
    scratch_types=[
        pltpu.VMEM((SEG, CH), jnp.int32),
        pltpu.VMEM((SEG, CH), jnp.int32),
        pltpu.VMEM((SEG, CH), jnp.int32),
        pltpu.VMEM((SEG, CH), jnp.int32),
        pltpu.VMEM((CH, D), jnp.float32),
        pltpu.VMEM((CH, D), jnp.float32),
        pltpu.VMEM((CH, D), jnp.float32),
        pltpu.VMEM((CH, D), jnp.float32),
        pltpu.VMEM_SHARED((NPAD, D), jnp.float32),
        pltpu.SemaphoreType.DMA,
        pltpu.SemaphoreType.DMA,
        pltpu.SemaphoreType.DMA,
        pltpu.SemaphoreType.DMA,
        pltpu.SemaphoreType.DMA,
        pltpu.SemaphoreType.DMA,
        pltpu.SemaphoreType.DMA,
        pltpu.SemaphoreType.DMA,
    ],
)
def _prop_kernel(hp_hbm, src_hbm, dst_hbm, out_hbm,
                 sseg0, sseg1, dseg0, dseg1, rows0, rows1, rows2, rows3,
                 acc, sem0, sem1, sem2, sem3, ssem0, ssem1, dsem0, dsem1):
  c = lax.axis_index("c")
  s = lax.axis_index("s")
  nch = jnp.where(c == 0, NCH0, NCH1)
  nseg = nch // SEG
  cbase = jnp.where(c == 0, s * NCH0, CHOFF1 + s * NCH1)
  rows = (rows0, rows1, rows2, rows3)
  sems = (sem0, sem1, sem2, sem3)
  ssegs = (sseg0, sseg1)
  dsegs = (dseg0, dseg1)
  ssems = (ssem0, ssem1)
  dsems = (dsem0, dsem1)

  zeros = jnp.zeros((16,), jnp.float32)

  def zbody(i, carry):
    for k in range(D // 16):
      rows0[i, pl.ds(k * 16, 16)] = zeros
    return carry

  lax.fori_loop(0, CH, zbody, 0)
  for k in range(ROWS_PER_TILE // CH):
    pltpu.sync_copy(rows0, acc.at[pl.ds(s * ROWS_PER_TILE + k * CH, CH)])

  @pl.when(nch > 0)
  def _():
    pltpu.sync_copy(src_hbm.at[pl.ds(cbase, SEG)], sseg0)
    pltpu.sync_copy(dst_hbm.at[pl.ds(cbase, SEG)], dseg0)

  plsc.subcore_barrier()

  @pl.when(nch > 0)
  def _():
    for b in range(NB):
      pltpu.async_copy(hp_hbm.at[sseg0.at[b]], rows[b], sems[b])

  def segment(g, ph):
    nxt = 1 - ph
    nxt_rows = cbase + (g + 1) * SEG

    @pl.when(g + 1 < nseg)
    def _():
      pltpu.async_copy(src_hbm.at[pl.ds(nxt_rows, SEG)], ssegs[nxt], ssems[nxt])
      pltpu.async_copy(dst_hbm.at[pl.ds(nxt_rows, SEG)], dsegs[nxt], dsems[nxt])

    for i in range(SEG):
      b = i % NB
      pltpu.make_async_copy(hp_hbm.at[ssegs[ph].at[i]], rows[b], sems[b]).wait()
      pltpu.sync_copy(rows[b], acc.at[dsegs[ph].at[i]], add=True)
      if i == XB:
        @pl.when(g + 1 < nseg)
        def _():
          pltpu.make_async_copy(src_hbm.at[pl.ds(nxt_rows, SEG)],
                                ssegs[nxt], ssems[nxt]).wait()
          pltpu.make_async_copy(dst_hbm.at[pl.ds(nxt_rows, SEG)],
                                dsegs[nxt], dsems[nxt]).wait()
      if i < XB:
        pltpu.async_copy(hp_hbm.at[ssegs[ph].at[i + NB]], rows[b], sems[b])
      else:
        @pl.when(g + 1 < nseg)
        def _():
          pltpu.async_copy(hp_hbm.at[ssegs[nxt].at[i + NB - SEG]],
                           rows[b], sems[b])

  def seg_pair(t, carry):
    segment(2 * t, 0)
    segment(2 * t + 1, 1)
    return carry

  lax.fori_loop(0, nseg // 2, seg_pair, 0)
  plsc.subcore_barrier()
  pltpu.sync_copy(acc.at[pl.ds(s * ROWS_PER_TILE, ROWS_PER_TILE)],
                  out_hbm.at[c, pl.ds(s * ROWS_PER_TILE, ROWS_PER_TILE)])


def _dis_from_hists(h_ref):
  deg = 1.0 + jnp.sum(h_ref[...], axis=0)
  return lax.rsqrt(deg)


def _tc_first(h_ref, x_ref, w_ref, o_ref):
  dis = _dis_from_hists(h_ref)
  h = jnp.dot(x_ref[...], w_ref[...], preferred_element_type=jnp.float32)
  o_ref[...] = h * dis[:, None]


def _tc_mid(h_ref, a0_ref, a1_ref, hp_ref, b_ref, w_ref, o_ref):
  dis = _dis_from_hists(h_ref)
  pre = (a0_ref[...] + a1_ref[...] + hp_ref[...]) * dis[:, None] + b_ref[...][None, :]
  x2 = jnp.maximum(pre, 0.0)
  rid = lax.broadcasted_iota(jnp.int32, (NPAD, 1), 0)
  x2 = jnp.where(rid < N_NODES, x2, 0.0)
  h = jnp.dot(x2, w_ref[...], preferred_element_type=jnp.float32)
  o_ref[...] = h * dis[:, None]


def _tc_last(h_ref, a0_ref, a1_ref, hp_ref, b_ref, o_ref):
  dis = _dis_from_hists(h_ref)
  o_ref[...] = (a0_ref[...] + a1_ref[...] + hp_ref[...]) * dis[:, None] + b_ref[...][None, :]


_f32 = jnp.float32
_tc_first_call = pl.pallas_call(
    _tc_first, out_shape=jax.ShapeDtypeStruct((NPAD, D), _f32))
_tc_mid_call = pl.pallas_call(
    _tc_mid, out_shape=jax.ShapeDtypeStruct((NPAD, D), _f32))
_tc_last_call = pl.pallas_call(
    _tc_last, out_shape=jax.ShapeDtypeStruct((NPAD, D), _f32))


@jax.jit
def kernel(x, edge_index, W1, b1, W2, b2):
  src = edge_index[0].astype(jnp.int32)
  dst = edge_index[1].astype(jnp.int32)
  n_edges = src.shape[0]
  pad = jnp.full((EPAD - n_edges,), N_NODES, jnp.int32)
  srcp = jnp.concatenate([src, pad])
  dstp = jnp.concatenate([dst, pad])
  src2 = srcp.reshape(EPAD // CH, CH)
  dst2 = dstp.reshape(EPAD // CH, CH)
  xp = jnp.pad(x, ((0, NPAD - N_NODES), (0, 0)))

  hists = _deg_kernel(dstp).reshape(NW, NPAD)
  hp1 = _tc_first_call(hists, xp, W1)
  acc1 = _prop_kernel(hp1, src2, dst2)
  hp2 = _tc_mid_call(hists, acc1[0], acc1[1], hp1, b1, W2)
  acc2 = _prop_kernel(hp2, src2, dst2)
  outp = _tc_last_call(hists, acc2[0], acc2[1], hp2, b2)
  return outp[:N_NODES]


# speedup vs baseline: 3.4525x; 3.4525x over previous
import functools

import jax
import jax.numpy as jnp
from jax import lax
from jax.experimental import pallas as pl
from jax.experimental.pallas import tpu as pltpu
from jax.experimental.pallas import tpu_sc as plsc

N_NODES = 10000
D = 128
NPAD = 10240
NC = 2
NS = 16
NW = NC * NS
CH = 64
TPW = NPAD
EPAD = NW * TPW
NCH0 = 160
NCH1 = 160
CHOFF1 = NS * NCH0
NB = 4
SEG = 16
XB = SEG - NB
ROWS_PER_TILE = NPAD // NS

_mesh = plsc.VectorSubcoreMesh(core_axis_name="c", subcore_axis_name="s")
_sc_params = pltpu.CompilerParams(needs_layout_passes=False)


@functools.partial(
    pl.kernel,
    out_type=jax.ShapeDtypeStruct((NC, NS, NPAD), jnp.float32),
    mesh=_mesh,
    compiler_params=_sc_params,
    scratch_types=[
        pltpu.VMEM((TPW,), jnp.int32),
        pltpu.VMEM((NPAD,), jnp.float32),
    ],
)
def _deg_kernel(dst_hbm, out_hbm, idx_v, hist_v):
  c = lax.axis_index("c")
  s = lax.axis_index("s")
  w = s * NC + c
  pltpu.sync_copy(dst_hbm.at[pl.ds(w * TPW, TPW)], idx_v)
  zeros = jnp.zeros((16,), jnp.float32)

  def zbody(i, carry):
    hist_v[pl.ds(i * 16, 16)] = zeros
    return carry

  lax.fori_loop(0, NPAD // 16, zbody, 0)
  ones = jnp.ones((16,), jnp.float32)

  def body(i, carry):
    idx = idx_v[pl.ds(i * 16, 16)]
    plsc.addupdate_scatter(hist_v, [idx], ones)
    return carry

  lax.fori_loop(0, TPW // 16, body, 0)
  pltpu.sync_copy(hist_v, out_hbm.at[c, s])


@functools.partial(
    pl.kernel,
    out_type=jax.ShapeDtypeStruct((NC, NPAD, D), jnp.float32),
    mesh=_mesh,
    compiler_params=_sc_params,
    scratch_types=[
        pltpu.VMEM((SEG, CH), jnp.int32),
        pltpu.VMEM((SEG, CH), jnp.int32),
        pltpu.VMEM((SEG, CH), jnp.int32),
        pltpu.VMEM((SEG, CH), jnp.int32),
        pltpu.VMEM((CH, D), jnp.float32),
        pltpu.VMEM((CH, D), jnp.float32),
        pltpu.VMEM((CH, D), jnp.float32),
        pltpu.VMEM((CH, D), jnp.float32),
        pltpu.VMEM_SHARED((NPAD, D), jnp.float32),
        pltpu.SemaphoreType.DMA,
        pltpu.SemaphoreType.DMA,
        pltpu.SemaphoreType.DMA,
        pltpu.SemaphoreType.DMA,
        pltpu.SemaphoreType.DMA,
        pltpu.SemaphoreType.DMA,
        pltpu.SemaphoreType.DMA,
        pltpu.SemaphoreType.DMA,
    ],
)
def _prop_kernel(hp_hbm, src_hbm, dst_hbm, out_hbm,
                 sseg0, sseg1, dseg0, dseg1, rows0, rows1, rows2, rows3,
                 acc, sem0, sem1, sem2, sem3, ssem0, ssem1, dsem0, dsem1):
  c = lax.axis_index("c")
  s = lax.axis_index("s")
  nch = jnp.where(c == 0, NCH0, NCH1)
  nseg = nch // SEG
  cbase = jnp.where(c == 0, s * NCH0, CHOFF1 + s * NCH1)
  rows = (rows0, rows1, rows2, rows3)
  sems = (sem0, sem1, sem2, sem3)
  ssegs = (sseg0, sseg1)
  dsegs = (dseg0, dseg1)
  ssems = (ssem0, ssem1)
  dsems = (dsem0, dsem1)

  zeros = jnp.zeros((16,), jnp.float32)

  def zbody(i, carry):
    for k in range(D // 16):
      rows0[i, pl.ds(k * 16, 16)] = zeros
    return carry

  lax.fori_loop(0, CH, zbody, 0)
  for k in range(ROWS_PER_TILE // CH):
    pltpu.sync_copy(rows0, acc.at[pl.ds(s * ROWS_PER_TILE + k * CH, CH)])

  @pl.when(nch > 0)
  def _():
    pltpu.sync_copy(src_hbm.at[pl.ds(cbase, SEG)], sseg0)
    pltpu.sync_copy(dst_hbm.at[pl.ds(cbase, SEG)], dseg0)

  plsc.subcore_barrier()

  @pl.when(nch > 0)
  def _():
    for b in range(NB):
      pltpu.async_copy(hp_hbm.at[sseg0.at[b]], rows[b], sems[b])

  def segment(g, ph):
    nxt = 1 - ph
    nxt_rows = cbase + (g + 1) * SEG

    @pl.when(g + 1 < nseg)
    def _():
      pltpu.async_copy(src_hbm.at[pl.ds(nxt_rows, SEG)], ssegs[nxt], ssems[nxt])
      pltpu.async_copy(dst_hbm.at[pl.ds(nxt_rows, SEG)], dsegs[nxt], dsems[nxt])

    for i in range(SEG):
      b = i % NB
      pltpu.make_async_copy(hp_hbm.at[ssegs[ph].at[i]], rows[b], sems[b]).wait()
      pltpu.sync_copy(rows[b], acc.at[dsegs[ph].at[i]], add=True)
      if i == XB:
        @pl.when(g + 1 < nseg)
        def _():
          pltpu.make_async_copy(src_hbm.at[pl.ds(nxt_rows, SEG)],
                                ssegs[nxt], ssems[nxt]).wait()
          pltpu.make_async_copy(dst_hbm.at[pl.ds(nxt_rows, SEG)],
                                dsegs[nxt], dsems[nxt]).wait()
      if i < XB:
        pltpu.async_copy(hp_hbm.at[ssegs[ph].at[i + NB]], rows[b], sems[b])
      else:
        @pl.when(g + 1 < nseg)
        def _():
          pltpu.async_copy(hp_hbm.at[ssegs[nxt].at[i + NB - SEG]],
                           rows[b], sems[b])

  def seg_pair(t, carry):
    segment(2 * t, 0)
    segment(2 * t + 1, 1)
    return carry

  lax.fori_loop(0, nseg // 2, seg_pair, 0)
  plsc.subcore_barrier()
  pltpu.sync_copy(acc.at[pl.ds(s * ROWS_PER_TILE, ROWS_PER_TILE)],
                  out_hbm.at[c, pl.ds(s * ROWS_PER_TILE, ROWS_PER_TILE)])


def _dis_from_hists(h_ref):
  deg = 1.0 + jnp.sum(h_ref[...], axis=0)
  return lax.rsqrt(deg)


def _tc_first(h_ref, x_ref, w_ref, o_ref):
  dis = _dis_from_hists(h_ref)
  h = jnp.dot(x_ref[...], w_ref[...], preferred_element_type=jnp.float32)
  o_ref[...] = h * dis[:, None]


def _tc_mid(h_ref, a0_ref, a1_ref, hp_ref, b_ref, w_ref, o_ref):
  dis = _dis_from_hists(h_ref)
  pre = (a0_ref[...] + a1_ref[...] + hp_ref[...]) * dis[:, None] + b_ref[...][None, :]
  x2 = jnp.maximum(pre, 0.0)
  rid = lax.broadcasted_iota(jnp.int32, (NPAD, 1), 0)
  x2 = jnp.where(rid < N_NODES, x2, 0.0)
  h = jnp.dot(x2, w_ref[...], preferred_element_type=jnp.float32)
  o_ref[...] = h * dis[:, None]


def _tc_last(h_ref, a0_ref, a1_ref, hp_ref, b_ref, o_ref):
  dis = _dis_from_hists(h_ref)
  o_ref[...] = (a0_ref[...] + a1_ref[...] + hp_ref[...]) * dis[:, None] + b_ref[...][None, :]


_f32 = jnp.float32
_tc_first_call = pl.pallas_call(
    _tc_first, out_shape=jax.ShapeDtypeStruct((NPAD, D), _f32))
_tc_mid_call = pl.pallas_call(
    _tc_mid, out_shape=jax.ShapeDtypeStruct((NPAD, D), _f32))
_tc_last_call = pl.pallas_call(
    _tc_last, out_shape=jax.ShapeDtypeStruct((NPAD, D), _f32))


@jax.jit
def kernel(x, edge_index, W1, b1, W2, b2):
  src = edge_index[0].astype(jnp.int32)
  dst = edge_index[1].astype(jnp.int32)
  n_edges = src.shape[0]
  pad = N_NODES + jnp.arange(EPAD - n_edges, dtype=jnp.int32) % (NPAD - N_NODES)
  srcp = jnp.concatenate([src, pad])
  dstp = jnp.concatenate([dst, pad])
  src2 = srcp.reshape(EPAD // CH, CH)
  dst2 = dstp.reshape(EPAD // CH, CH)
  xp = jnp.pad(x, ((0, NPAD - N_NODES), (0, 0)))

  hists = _deg_kernel(dstp).reshape(NW, NPAD)
  hp1 = _tc_first_call(hists, xp, W1)
  acc1 = _prop_kernel(hp1, src2, dst2)
  hp2 = _tc_mid_call(hists, acc1[0], acc1[1], hp1, b1, W2)
  acc2 = _prop_kernel(hp2, src2, dst2)
  outp = _tc_last_call(hists, acc2[0], acc2[1], hp2, b2)
  return outp[:N_NODES]


# speedup vs baseline: 3.5047x; 1.0151x over previous
import functools

import jax
import jax.numpy as jnp
from jax import lax
from jax.experimental import pallas as pl
from jax.experimental.pallas import tpu as pltpu
from jax.experimental.pallas import tpu_sc as plsc

N_NODES = 10000
D = 128
NPAD = 10240
NC = 2
NS = 16
NW = NC * NS
CH = 64
TPW = NPAD
EPAD = NW * TPW
NCH0 = 160
NCH1 = 160
CHOFF1 = NS * NCH0
NB = 4
SEG = 16
XB = SEG - NB
ROWS_PER_TILE = NPAD // NS

_mesh = plsc.VectorSubcoreMesh(core_axis_name="c", subcore_axis_name="s")
_sc_params = pltpu.CompilerParams(needs_layout_passes=False)


@functools.partial(
    pl.kernel,
    out_type=jax.ShapeDtypeStruct((NC, NS, NPAD), jnp.float32),
    mesh=_mesh,
    compiler_params=_sc_params,
    scratch_types=[
        pltpu.VMEM((TPW,), jnp.int32),
        pltpu.VMEM((NPAD,), jnp.float32),
    ],
)
def _deg_kernel(dst_hbm, out_hbm, idx_v, hist_v):
  c = lax.axis_index("c")
  s = lax.axis_index("s")
  w = s * NC + c
  pltpu.sync_copy(dst_hbm.at[pl.ds(w * TPW, TPW)], idx_v)
  zeros = jnp.zeros((16,), jnp.float32)

  def zbody(i, carry):
    hist_v[pl.ds(i * 16, 16)] = zeros
    return carry

  lax.fori_loop(0, NPAD // 16, zbody, 0)
  ones = jnp.ones((16,), jnp.float32)

  def body(i, carry):
    idx = idx_v[pl.ds(i * 16, 16)]
    plsc.addupdate_scatter(hist_v, [idx], ones)
    return carry

  lax.fori_loop(0, TPW // 16, body, 0)
  pltpu.sync_copy(hist_v, out_hbm.at[c, s])


@functools.partial(
    pl.kernel,
    out_type=jax.ShapeDtypeStruct((NC, NPAD, D), jnp.float32),
    mesh=_mesh,
    compiler_params=_sc_params,
    scratch_types=[
        pltpu.VMEM((SEG, CH), jnp.int32),
        pltpu.VMEM((SEG, CH), jnp.int32),
        pltpu.VMEM((SEG, CH), jnp.int32),
        pltpu.VMEM((SEG, CH), jnp.int32),
        pltpu.VMEM((CH, D), jnp.float32),
        pltpu.VMEM((CH, D), jnp.float32),
        pltpu.VMEM((CH, D), jnp.float32),
        pltpu.VMEM((CH, D), jnp.float32),
        pltpu.VMEM_SHARED((NPAD, D), jnp.float32),
        pltpu.SemaphoreType.DMA,
        pltpu.SemaphoreType.DMA,
        pltpu.SemaphoreType.DMA,
        pltpu.SemaphoreType.DMA,
        pltpu.SemaphoreType.DMA,
        pltpu.SemaphoreType.DMA,
        pltpu.SemaphoreType.DMA,
        pltpu.SemaphoreType.DMA,
    ],
)
def _prop_kernel(hp_hbm, src_hbm, dst_hbm, out_hbm,
                 sseg0, sseg1, dseg0, dseg1, rows0, rows1, rows2, rows3,
                 acc, sem0, sem1, sem2, sem3, ssem0, ssem1, dsem0, dsem1):
  c = lax.axis_index("c")
  s = lax.axis_index("s")
  nch = jnp.where(c == 0, NCH0, NCH1)
  nseg = nch // SEG
  cbase = jnp.where(c == 0, s * NCH0, CHOFF1 + s * NCH1)
  rows = (rows0, rows1, rows2, rows3)
  sems = (sem0, sem1, sem2, sem3)
  ssegs = (sseg0, sseg1)
  dsegs = (dseg0, dseg1)
  ssems = (ssem0, ssem1)
  dsems = (dsem0, dsem1)

  zeros = jnp.zeros((16,), jnp.float32)

  def zbody(i, carry):
    for k in range(D // 16):
      rows0[i, pl.ds(k * 16, 16)] = zeros
    return carry

  lax.fori_loop(0, CH, zbody, 0)
  for k in range(ROWS_PER_TILE // CH):
    pltpu.sync_copy(rows0, acc.at[pl.ds(s * ROWS_PER_TILE + k * CH, CH)])

  @pl.when(nch > 0)
  def _():
    pltpu.sync_copy(src_hbm.at[pl.ds(cbase, SEG)], sseg0)
    pltpu.sync_copy(dst_hbm.at[pl.ds(cbase, SEG)], dseg0)

  plsc.subcore_barrier()

  @pl.when(nch > 0)
  def _():
    for b in range(NB):
      pltpu.async_copy(hp_hbm.at[sseg0.at[b]], rows[b], sems[b])

  def segment(g, ph):
    nxt = 1 - ph
    nxt_rows = cbase + (g + 1) * SEG

    @pl.when(g + 1 < nseg)
    def _():
      pltpu.async_copy(src_hbm.at[pl.ds(nxt_rows, SEG)], ssegs[nxt], ssems[nxt])
      pltpu.async_copy(dst_hbm.at[pl.ds(nxt_rows, SEG)], dsegs[nxt], dsems[nxt])

    for i in range(SEG):
      b = i % NB
      pltpu.make_async_copy(hp_hbm.at[ssegs[ph].at[i]], rows[b], sems[b]).wait()
      pltpu.sync_copy(rows[b], acc.at[dsegs[ph].at[i]], add=True)
      if i == XB:
        @pl.when(g + 1 < nseg)
        def _():
          pltpu.make_async_copy(src_hbm.at[pl.ds(nxt_rows, SEG)],
                                ssegs[nxt], ssems[nxt]).wait()
          pltpu.make_async_copy(dst_hbm.at[pl.ds(nxt_rows, SEG)],
                                dsegs[nxt], dsems[nxt]).wait()
      if i < XB:
        pltpu.async_copy(hp_hbm.at[ssegs[ph].at[i + NB]], rows[b], sems[b])
      else:
        @pl.when(g + 1 < nseg)
        def _():
          pltpu.async_copy(hp_hbm.at[ssegs[nxt].at[i + NB - SEG]],
                           rows[b], sems[b])

  def seg_pair(t, carry):
    segment(2 * t, 0)
    segment(2 * t + 1, 1)
    return carry

  lax.fori_loop(0, nseg // 2, seg_pair, 0)
  plsc.subcore_barrier()
  pltpu.sync_copy(acc.at[pl.ds(s * ROWS_PER_TILE, ROWS_PER_TILE)],
                  out_hbm.at[c, pl.ds(s * ROWS_PER_TILE, ROWS_PER_TILE)])


def _dis_from_hists(h_ref):
  deg = 1.0 + jnp.sum(h_ref[...], axis=0)
  return lax.rsqrt(deg)


def _tc_first(h_ref, x_ref, w_ref, o_ref):
  dis = _dis_from_hists(h_ref)
  h = jnp.dot(x_ref[...], w_ref[...], preferred_element_type=jnp.float32)
  o_ref[...] = h * dis[:, None]


def _tc_mid(h_ref, a0_ref, a1_ref, hp_ref, b_ref, w_ref, o_ref):
  dis = _dis_from_hists(h_ref)
  pre = (a0_ref[...] + a1_ref[...] + hp_ref[...]) * dis[:, None] + b_ref[...][None, :]
  x2 = jnp.maximum(pre, 0.0)
  rid = lax.broadcasted_iota(jnp.int32, (NPAD, 1), 0)
  x2 = jnp.where(rid < N_NODES, x2, 0.0)
  h = jnp.dot(x2, w_ref[...], preferred_element_type=jnp.float32)
  o_ref[...] = h * dis[:, None]


def _tc_last(h_ref, a0_ref, a1_ref, hp_ref, b_ref, o_ref):
  dis = _dis_from_hists(h_ref)
  full = (a0_ref[...] + a1_ref[...] + hp_ref[...]) * dis[:, None] + b_ref[...][None, :]
  o_ref[...] = full[:N_NODES]


_f32 = jnp.float32
_tc_first_call = pl.pallas_call(
    _tc_first, out_shape=jax.ShapeDtypeStruct((NPAD, D), _f32))
_tc_mid_call = pl.pallas_call(
    _tc_mid, out_shape=jax.ShapeDtypeStruct((NPAD, D), _f32))
_tc_last_call = pl.pallas_call(
    _tc_last, out_shape=jax.ShapeDtypeStruct((N_NODES, D), _f32))


@jax.jit
def kernel(x, edge_index, W1, b1, W2, b2):
  src = edge_index[0].astype(jnp.int32)
  dst = edge_index[1].astype(jnp.int32)
  n_edges = src.shape[0]
  pad = N_NODES + jnp.arange(EPAD - n_edges, dtype=jnp.int32) % (NPAD - N_NODES)
  srcp = jnp.concatenate([src, pad])
  dstp = jnp.concatenate([dst, pad])
  src2 = srcp.reshape(EPAD // CH, CH)
  dst2 = dstp.reshape(EPAD // CH, CH)
  xp = jnp.pad(x, ((0, NPAD - N_NODES), (0, 0)))

  hists = _deg_kernel(dstp).reshape(NW, NPAD)
  hp1 = _tc_first_call(hists, xp, W1)
  acc1 = _prop_kernel(hp1, src2, dst2)
  hp2 = _tc_mid_call(hists, acc1[0], acc1[1], hp1, b1, W2)
  acc2 = _prop_kernel(hp2, src2, dst2)
  return _tc_last_call(hists, acc2[0], acc2[1], hp2, b2)
